# Initial kernel scaffold; baseline (speedup 1.0000x reference)
import jax
import jax.numpy as jnp
from jax.experimental import pallas as pl


def kernel(y_true, y_pred):
    raise NotImplementedError("write your pallas kernel here")


# speedup vs baseline: 67.7927x; 67.7927x over previous
import functools

import jax
import jax.numpy as jnp
from jax import lax
from jax.experimental import pallas as pl
from jax.experimental.pallas import tpu as pltpu
from jax.experimental.pallas import tpu_sc as plsc

B, S, V = 8, 32, 100000
ROWS = B * S
LANES = 16
NUM_WORKERS = 32
ROWS_PER_WORKER = ROWS // NUM_WORKERS
NUM_GROUPS = V // LANES


def _sc_body(x_hbm, lab_hbm, out_hbm, row_v, lab_v, res_v):
    cid = lax.axis_index("c")
    sid = lax.axis_index("s")
    wid = sid * 2 + cid
    base = wid * ROWS_PER_WORKER
    lane = lax.iota(jnp.int32, LANES)
    pltpu.sync_copy(lab_hbm, lab_v)
    acc = jnp.float32(0.0)
    for i in range(ROWS_PER_WORKER):
        row = base + i
        pltpu.sync_copy(x_hbm.at[row], row_v)
        g0 = row // LANES
        rl = row - g0 * LANES
        lvec = lab_v[pl.ds(g0 * LANES, LANES)]
        lab = jnp.sum(jnp.where(lane == rl, lvec, 0))
        g_lab = lab // LANES
        r = lab - g_lab * LANES
        ab = row_v[pl.ds(g_lab * LANES, LANES)].astype(jnp.int32)
        v_splat = jnp.broadcast_to(jnp.sum(jnp.where(lane == r, ab, 0)), (LANES,))

        def body_lo(g, cnt):
            a = row_v[pl.ds(g * LANES, LANES)].astype(jnp.int32)
            return cnt + (a >= v_splat).astype(jnp.int32)

        def body_hi(g, cnt):
            a = row_v[pl.ds(g * LANES, LANES)].astype(jnp.int32)
            return cnt + (a > v_splat).astype(jnp.int32)

        zeros = jnp.zeros((LANES,), jnp.int32)
        cnt_lo = lax.fori_loop(0, g_lab, body_lo, zeros)
        cnt_hi = lax.fori_loop(g_lab + 1, NUM_GROUPS, body_hi, zeros)
        mb = (ab > v_splat) | ((ab == v_splat) & (lane < r))
        total = jnp.sum(cnt_lo) + jnp.sum(cnt_hi) + jnp.sum(mb.astype(jnp.int32))
        acc = acc + jnp.where(total <= 4, jnp.float32(1.0 / ROWS), jnp.float32(0.0))
    res_v[...] = jnp.broadcast_to(acc, (LANES,))
    pltpu.sync_copy(res_v, out_hbm.at[wid])


_sc_count = functools.partial(
    pl.kernel,
    out_type=jax.ShapeDtypeStruct((NUM_WORKERS, LANES), jnp.float32),
    mesh=plsc.VectorSubcoreMesh(core_axis_name="c", subcore_axis_name="s"),
    scratch_types=[
        pltpu.VMEM((V,), jnp.float32),
        pltpu.VMEM((ROWS,), jnp.int32),
        pltpu.VMEM((LANES,), jnp.float32),
    ],
    compiler_params=pltpu.CompilerParams(needs_layout_passes=False),
)(_sc_body)


def _tc_combine(p_ref, o_ref):
    o_ref[0, 0] = jnp.sum(p_ref[...]) * jnp.float32(1.0 / LANES)


def kernel(y_true, y_pred):
    labels = y_true.astype(jnp.int32).reshape(ROWS)
    x = y_pred.reshape(ROWS, V)
    partials = _sc_count(x, labels)
    value2d = pl.pallas_call(
        _tc_combine,
        out_shape=jax.ShapeDtypeStruct((1, 1), jnp.float32),
        in_specs=[pl.BlockSpec(memory_space=pltpu.VMEM)],
        out_specs=pl.BlockSpec(memory_space=pltpu.SMEM),
    )(partials)
    return (y_pred, value2d.reshape(()))


# speedup vs baseline: 112.4375x; 1.6585x over previous
import functools

import jax
import jax.numpy as jnp
from jax import lax
from jax.experimental import pallas as pl
from jax.experimental.pallas import tpu as pltpu
from jax.experimental.pallas import tpu_sc as plsc

B, S, V = 8, 32, 100000
ROWS = B * S
LANES = 16
NUM_WORKERS = 32
ROWS_PER_WORKER = ROWS // NUM_WORKERS
NUM_GROUPS = V // LANES


def _sc_body(x_hbm, lab_hbm, out_hbm, row_v, lab_v, res_v):
    cid = lax.axis_index("c")
    sid = lax.axis_index("s")
    wid = sid * 2 + cid
    base = wid * ROWS_PER_WORKER
    lane = lax.iota(jnp.int32, LANES)
    pltpu.sync_copy(lab_hbm, lab_v)
    acc = jnp.float32(0.0)
    for i in range(ROWS_PER_WORKER):
        row = base + i
        pltpu.sync_copy(x_hbm.at[row], row_v)
        g0 = row // LANES
        rl = row - g0 * LANES
        lvec = lab_v[pl.ds(g0 * LANES, LANES)]
        lab = jnp.sum(jnp.where(lane == rl, lvec, 0))
        g_lab = lab // LANES
        r = lab - g_lab * LANES
        ab = row_v[pl.ds(g_lab * LANES, LANES)].astype(jnp.int32)
        v_splat = jnp.broadcast_to(jnp.sum(jnp.where(lane == r, ab, 0)), (LANES,))

        def body_lo(g, cnt):
            a = row_v[pl.ds(g, LANES)].astype(jnp.int32)
            return cnt + (a >= v_splat).astype(jnp.int32)

        def body_hi(g, cnt):
            a = row_v[pl.ds(g, LANES)].astype(jnp.int32)
            return cnt + (a > v_splat).astype(jnp.int32)

        zeros = jnp.zeros((LANES,), jnp.int32)
        cnt_lo = plsc.parallel_loop(
            0, g_lab * LANES, LANES, unroll=8, carry=zeros
        )(body_lo)
        cnt_hi = plsc.parallel_loop(
            (g_lab + 1) * LANES, V, LANES, unroll=8, carry=zeros
        )(body_hi)
        mb = (ab > v_splat) | ((ab == v_splat) & (lane < r))
        total = jnp.sum(cnt_lo) + jnp.sum(cnt_hi) + jnp.sum(mb.astype(jnp.int32))
        acc = acc + jnp.where(total <= 4, jnp.float32(1.0 / ROWS), jnp.float32(0.0))
    res_v[...] = jnp.broadcast_to(acc, (LANES,))
    pltpu.sync_copy(res_v, out_hbm.at[wid])


_sc_count = functools.partial(
    pl.kernel,
    out_type=jax.ShapeDtypeStruct((NUM_WORKERS, LANES), jnp.float32),
    mesh=plsc.VectorSubcoreMesh(core_axis_name="c", subcore_axis_name="s"),
    scratch_types=[
        pltpu.VMEM((V,), jnp.float32),
        pltpu.VMEM((ROWS,), jnp.int32),
        pltpu.VMEM((LANES,), jnp.float32),
    ],
    compiler_params=pltpu.CompilerParams(needs_layout_passes=False),
)(_sc_body)


def _tc_combine(p_ref, o_ref):
    o_ref[0, 0] = jnp.sum(p_ref[...]) * jnp.float32(1.0 / LANES)


def kernel(y_true, y_pred):
    labels = y_true.astype(jnp.int32).reshape(ROWS)
    x = y_pred.reshape(ROWS, V)
    partials = _sc_count(x, labels)
    value2d = pl.pallas_call(
        _tc_combine,
        out_shape=jax.ShapeDtypeStruct((1, 1), jnp.float32),
        in_specs=[pl.BlockSpec(memory_space=pltpu.VMEM)],
        out_specs=pl.BlockSpec(memory_space=pltpu.SMEM),
    )(partials)
    return (y_pred, value2d.reshape(()))
